# Initial kernel scaffold; baseline (speedup 1.0000x reference)
import jax
import jax.numpy as jnp
from jax import lax
from jax.experimental import pallas as pl
from jax.experimental.pallas import tpu as pltpu
from jax.experimental.pallas import tpu_sc as plsc

N_ROWS = 100000
IN_FEATURES = 256
OUT_FEATURES = 128
LEVEL_W = 32
SEL_W = 16
VROWS = N_ROWS * (IN_FEATURES // LEVEL_W)
NUM_WORKERS = 32
ROWS_PER_WORKER = VROWS // NUM_WORKERS


def _body(h_hbm, out_hbm):
    c = lax.axis_index("c")
    s = lax.axis_index("s")
    wid = s * 2 + c
    base = wid * ROWS_PER_WORKER
    pltpu.sync_copy(
        h_hbm.at[pl.ds(base, ROWS_PER_WORKER), pl.ds(0, SEL_W)],
        out_hbm.at[pl.ds(base, ROWS_PER_WORKER), :],
    )


@jax.jit
def kernel(h):
    h2 = h.reshape(VROWS, LEVEL_W)
    mesh = plsc.VectorSubcoreMesh(core_axis_name="c", subcore_axis_name="s")
    out2 = pl.kernel(
        _body,
        out_type=jax.ShapeDtypeStruct((VROWS, SEL_W), jnp.float32),
        mesh=mesh,
    )(h2)
    return out2.reshape(N_ROWS, OUT_FEATURES)


# speedup vs baseline: 1.0007x; 1.0007x over previous
import jax
import jax.numpy as jnp
from jax import lax
from jax.experimental import pallas as pl
from jax.experimental.pallas import tpu as pltpu
from jax.experimental.pallas import tpu_sc as plsc

N_ROWS = 100000
IN_FEATURES = 256
OUT_FEATURES = 128
N_LEVELS = 8
SEL_W = 16
LEVEL_W = 32
NUM_WORKERS = 32
CHUNK_ROWS = 160
NCHUNKS = N_ROWS // CHUNK_ROWS


def _body(h_hbm, out_hbm, in_buf, out_buf):
    c = lax.axis_index("c")
    s = lax.axis_index("s")
    w = s * 2 + c
    nc = (NCHUNKS - 1 - w) // NUM_WORKERS + 1

    def chunk_body(k, carry):
        chunk = w + k * NUM_WORKERS
        row0 = chunk * CHUNK_ROWS
        pltpu.sync_copy(h_hbm.at[pl.ds(row0, CHUNK_ROWS), :], in_buf)

        def row_body(r, carry2):
            for l in range(N_LEVELS):
                out_buf[r, pl.ds(SEL_W * l, SEL_W)] = in_buf[
                    r, pl.ds(LEVEL_W * l, SEL_W)
                ]
            return carry2

        lax.fori_loop(0, CHUNK_ROWS, row_body, 0)
        pltpu.sync_copy(out_buf, out_hbm.at[pl.ds(row0, CHUNK_ROWS), :])
        return carry

    lax.fori_loop(0, nc, chunk_body, 0)


@jax.jit
def kernel(h):
    mesh = plsc.VectorSubcoreMesh(core_axis_name="c", subcore_axis_name="s")
    return pl.kernel(
        _body,
        out_type=jax.ShapeDtypeStruct((N_ROWS, OUT_FEATURES), jnp.float32),
        mesh=mesh,
        scratch_types=[
            pltpu.VMEM((CHUNK_ROWS, IN_FEATURES), jnp.float32),
            pltpu.VMEM((CHUNK_ROWS, OUT_FEATURES), jnp.float32),
        ],
    )(h)


# speedup vs baseline: 1.5312x; 1.5300x over previous
import jax
import jax.numpy as jnp
from jax import lax
from jax.experimental import pallas as pl
from jax.experimental.pallas import tpu as pltpu
from jax.experimental.pallas import tpu_sc as plsc

N_ROWS = 100000
IN_FEATURES = 256
OUT_FEATURES = 128
N_LEVELS = 8
SEL_W = 16
LEVEL_W = 32
NUM_WORKERS = 32
N_BLOCKS = N_ROWS // 8
CB = 20
CHUNK_ROWS = CB * 8
NCH = 20


def _body(h_hbm, out_hbm, in0, in1, out0, out1, si0, si1, so0, so1):
    c = lax.axis_index("c")
    s = lax.axis_index("s")
    w = s * 2 + c
    bstart = (N_BLOCKS * w) // NUM_WORKERS
    bend = (N_BLOCKS * (w + 1)) // NUM_WORKERS
    ins = (in0, in1)
    outs = (out0, out1)
    sis = (si0, si1)
    sos = (so0, so1)

    def row0_of(k):
        return jnp.minimum(bstart + k * CB, bend - CB) * 8

    def in_copy(b, r0):
        return pltpu.make_async_copy(
            h_hbm.at[pl.ds(r0, CHUNK_ROWS), :], ins[b], sis[b]
        )

    def out_copy(b, r0):
        return pltpu.make_async_copy(
            outs[b], out_hbm.at[pl.ds(r0, CHUNK_ROWS), :], sos[b]
        )

    for b in range(2):
        in_copy(b, row0_of(b)).start()

    def outer(j, carry):
        for b in range(2):
            k = j * 2 + b
            r0 = row0_of(k)
            in_copy(b, r0).wait()

            @pl.when(k >= 2)
            def _():
                out_copy(b, row0_of(k - 2)).wait()

            def row_body(r, cc):
                for l in range(N_LEVELS):
                    outs[b][r, pl.ds(SEL_W * l, SEL_W)] = ins[b][
                        r, pl.ds(LEVEL_W * l, SEL_W)
                    ]
                return cc

            lax.fori_loop(0, CHUNK_ROWS, row_body, 0)

            @pl.when(k == NCH - 1)
            def _():
                out_copy(1 - b, row0_of(k - 1)).wait()

            out_copy(b, r0).start()

            @pl.when(k + 2 < NCH)
            def _():
                in_copy(b, row0_of(k + 2)).start()

        return carry

    lax.fori_loop(0, NCH // 2, outer, 0)
    out_copy(1, row0_of(NCH - 1)).wait()


@jax.jit
def kernel(h):
    mesh = plsc.VectorSubcoreMesh(core_axis_name="c", subcore_axis_name="s")
    return pl.kernel(
        _body,
        out_type=jax.ShapeDtypeStruct((N_ROWS, OUT_FEATURES), jnp.float32),
        mesh=mesh,
        scratch_types=[
            pltpu.VMEM((CHUNK_ROWS, IN_FEATURES), jnp.float32),
            pltpu.VMEM((CHUNK_ROWS, IN_FEATURES), jnp.float32),
            pltpu.VMEM((CHUNK_ROWS, OUT_FEATURES), jnp.float32),
            pltpu.VMEM((CHUNK_ROWS, OUT_FEATURES), jnp.float32),
            pltpu.SemaphoreType.DMA,
            pltpu.SemaphoreType.DMA,
            pltpu.SemaphoreType.DMA,
            pltpu.SemaphoreType.DMA,
        ],
    )(h)


# speedup vs baseline: 1.5360x; 1.0031x over previous
import jax
import jax.numpy as jnp
from jax import lax
from jax.experimental import pallas as pl
from jax.experimental.pallas import tpu as pltpu
from jax.experimental.pallas import tpu_sc as plsc

N_ROWS = 100000
IN_FEATURES = 256
OUT_FEATURES = 128
N_LEVELS = 8
SEL_W = 16
LEVEL_W = 32
NUM_WORKERS = 32
N_BLOCKS = N_ROWS // 8
CB = 20
CHUNK_ROWS = CB * 8
NCH = 20


def _body(h_hbm, out_hbm, in0, in1, out0, out1, si0, si1, so0, so1):
    c = lax.axis_index("c")
    s = lax.axis_index("s")
    w = s * 2 + c
    bstart = (N_BLOCKS * w) // NUM_WORKERS
    bend = (N_BLOCKS * (w + 1)) // NUM_WORKERS
    ins = (in0, in1)
    outs = (out0, out1)
    sis = (si0, si1)
    sos = (so0, so1)

    def row0_of(k):
        return jnp.minimum(bstart + k * CB, bend - CB) * 8

    def in_copy(b, r0):
        return pltpu.make_async_copy(
            h_hbm.at[pl.ds(r0, CHUNK_ROWS), :], ins[b], sis[b]
        )

    def out_copy(b, r0):
        return pltpu.make_async_copy(
            outs[b], out_hbm.at[pl.ds(r0, CHUNK_ROWS), :], sos[b]
        )

    for b in range(2):
        in_copy(b, row0_of(b)).start()

    def outer(j, carry):
        for b in range(2):
            k = j * 2 + b
            r0 = row0_of(k)
            in_copy(b, r0).wait()

            @pl.when(k >= 2)
            def _():
                out_copy(b, row0_of(k - 2)).wait()

            def row_body(r, cc):
                for l in range(N_LEVELS):
                    outs[b][r, pl.ds(SEL_W * l, SEL_W)] = ins[b][
                        r, pl.ds(LEVEL_W * l, SEL_W)
                    ]
                return cc

            lax.fori_loop(0, CHUNK_ROWS, row_body, 0, unroll=8)

            @pl.when(k == NCH - 1)
            def _():
                out_copy(1 - b, row0_of(k - 1)).wait()

            out_copy(b, r0).start()

            @pl.when(k + 2 < NCH)
            def _():
                in_copy(b, row0_of(k + 2)).start()

        return carry

    lax.fori_loop(0, NCH // 2, outer, 0)
    out_copy(1, row0_of(NCH - 1)).wait()


@jax.jit
def kernel(h):
    mesh = plsc.VectorSubcoreMesh(core_axis_name="c", subcore_axis_name="s")
    return pl.kernel(
        _body,
        out_type=jax.ShapeDtypeStruct((N_ROWS, OUT_FEATURES), jnp.float32),
        mesh=mesh,
        scratch_types=[
            pltpu.VMEM((CHUNK_ROWS, IN_FEATURES), jnp.float32),
            pltpu.VMEM((CHUNK_ROWS, IN_FEATURES), jnp.float32),
            pltpu.VMEM((CHUNK_ROWS, OUT_FEATURES), jnp.float32),
            pltpu.VMEM((CHUNK_ROWS, OUT_FEATURES), jnp.float32),
            pltpu.SemaphoreType.DMA,
            pltpu.SemaphoreType.DMA,
            pltpu.SemaphoreType.DMA,
            pltpu.SemaphoreType.DMA,
        ],
    )(h)


# speedup vs baseline: 2.1126x; 1.3754x over previous
import jax
import jax.numpy as jnp
from jax import lax
from jax.experimental import pallas as pl
from jax.experimental.pallas import tpu as pltpu
from jax.experimental.pallas import tpu_sc as plsc

N_ROWS = 100000
IN_FEATURES = 256
OUT_FEATURES = 128
N_LEVELS = 8
SEL_W = 16
LEVEL_W = 32
NUM_WORKERS = 32
N_BLOCKS = N_ROWS // 8
CB = 30
CHUNK_ROWS = CB * 8
NCH = 14


def _body(h_hbm, out_hbm, buf0, buf1, si0, si1, so0, so1):
    c = lax.axis_index("c")
    s = lax.axis_index("s")
    w = s * 2 + c
    bstart = (N_BLOCKS * w) // NUM_WORKERS
    bend = (N_BLOCKS * (w + 1)) // NUM_WORKERS
    bufs = (buf0, buf1)
    sis = (si0, si1)
    sos = (so0, so1)

    def row0_of(k):
        return jnp.minimum(bstart + k * CB, bend - CB) * 8

    def in_copy(b, r0):
        return pltpu.make_async_copy(
            h_hbm.at[pl.ds(r0, CHUNK_ROWS), :], bufs[b], sis[b]
        )

    def out_copy(b, r0):
        return pltpu.make_async_copy(
            bufs[b].at[:, pl.ds(0, OUT_FEATURES)],
            out_hbm.at[pl.ds(r0, CHUNK_ROWS), :],
            sos[b],
        )

    for b in range(2):
        in_copy(b, row0_of(b)).start()

    def step(k, b):
        in_copy(b, row0_of(k)).wait()

        @pl.when(k >= 1)
        def _():
            out_copy(1 - b, row0_of(k - 1)).wait()

        @pl.when(jnp.logical_and(k >= 1, k + 1 < NCH))
        def _():
            in_copy(1 - b, row0_of(k + 1)).start()

        def row_body(r, cc):
            for l in range(N_LEVELS):
                bufs[b][r, pl.ds(SEL_W * l, SEL_W)] = bufs[b][
                    r, pl.ds(LEVEL_W * l, SEL_W)
                ]
            return cc

        lax.fori_loop(0, CHUNK_ROWS, row_body, 0, unroll=8)
        out_copy(b, row0_of(k)).start()

    def outer(j, carry):
        step(j * 2, 0)
        step(j * 2 + 1, 1)
        return carry

    lax.fori_loop(0, NCH // 2, outer, 0)
    out_copy(1, row0_of(NCH - 1)).wait()


@jax.jit
def kernel(h):
    mesh = plsc.VectorSubcoreMesh(core_axis_name="c", subcore_axis_name="s")
    return pl.kernel(
        _body,
        out_type=jax.ShapeDtypeStruct((N_ROWS, OUT_FEATURES), jnp.float32),
        mesh=mesh,
        scratch_types=[
            pltpu.VMEM((CHUNK_ROWS, IN_FEATURES), jnp.float32),
            pltpu.VMEM((CHUNK_ROWS, IN_FEATURES), jnp.float32),
            pltpu.SemaphoreType.DMA,
            pltpu.SemaphoreType.DMA,
            pltpu.SemaphoreType.DMA,
            pltpu.SemaphoreType.DMA,
        ],
    )(h)


# speedup vs baseline: 2.2078x; 1.0451x over previous
import jax
import jax.numpy as jnp
from jax import lax
from jax.experimental import pallas as pl
from jax.experimental.pallas import tpu as pltpu
from jax.experimental.pallas import tpu_sc as plsc

N_ROWS = 100000
IN_FEATURES = 256
OUT_FEATURES = 128
N_LEVELS = 8
SEL_W = 16
LEVEL_W = 32
NUM_WORKERS = 32
N_BLOCKS = N_ROWS // 8
CB = 28
CHUNK_ROWS = CB * 8
NCH = 14


def _body(h_hbm, out_hbm, buf0, buf1, si0, si1, so0, so1):
    c = lax.axis_index("c")
    s = lax.axis_index("s")
    w = s * 2 + c
    bstart = (N_BLOCKS * w) // NUM_WORKERS
    bend = (N_BLOCKS * (w + 1)) // NUM_WORKERS
    bufs = (buf0, buf1)
    sis = (si0, si1)
    sos = (so0, so1)

    def row0_of(k):
        return jnp.minimum(bstart + k * CB, bend - CB) * 8

    def in_copy(b, r0):
        return pltpu.make_async_copy(
            h_hbm.at[pl.ds(r0, CHUNK_ROWS), :], bufs[b], sis[b]
        )

    def out_copy(b, r0):
        return pltpu.make_async_copy(
            bufs[b].at[:, pl.ds(0, OUT_FEATURES)],
            out_hbm.at[pl.ds(r0, CHUNK_ROWS), :],
            sos[b],
        )

    for b in range(2):
        in_copy(b, row0_of(b)).start()

    def step(k, b):
        in_copy(b, row0_of(k)).wait()

        @pl.when(k >= 1)
        def _():
            out_copy(1 - b, row0_of(k - 1)).wait()

        @pl.when(jnp.logical_and(k >= 1, k + 1 < NCH))
        def _():
            in_copy(1 - b, row0_of(k + 1)).start()

        def row_body(r, cc):
            for l in range(N_LEVELS):
                bufs[b][r, pl.ds(SEL_W * l, SEL_W)] = bufs[b][
                    r, pl.ds(LEVEL_W * l, SEL_W)
                ]
            return cc

        lax.fori_loop(0, CHUNK_ROWS, row_body, 0, unroll=8)
        out_copy(b, row0_of(k)).start()

    def outer(j, carry):
        step(j * 2, 0)
        step(j * 2 + 1, 1)
        return carry

    lax.fori_loop(0, NCH // 2, outer, 0)
    out_copy(1, row0_of(NCH - 1)).wait()


@jax.jit
def kernel(h):
    mesh = plsc.VectorSubcoreMesh(core_axis_name="c", subcore_axis_name="s")
    return pl.kernel(
        _body,
        out_type=jax.ShapeDtypeStruct((N_ROWS, OUT_FEATURES), jnp.float32),
        mesh=mesh,
        scratch_types=[
            pltpu.VMEM((CHUNK_ROWS, IN_FEATURES), jnp.float32),
            pltpu.VMEM((CHUNK_ROWS, IN_FEATURES), jnp.float32),
            pltpu.SemaphoreType.DMA,
            pltpu.SemaphoreType.DMA,
            pltpu.SemaphoreType.DMA,
            pltpu.SemaphoreType.DMA,
        ],
    )(h)


# speedup vs baseline: 2.2359x; 1.0127x over previous
import jax
import jax.numpy as jnp
from jax import lax
from jax.experimental import pallas as pl
from jax.experimental.pallas import tpu as pltpu
from jax.experimental.pallas import tpu_sc as plsc

N_ROWS = 100000
IN_FEATURES = 256
OUT_FEATURES = 128
N_LEVELS = 8
SEL_W = 16
LEVEL_W = 32
NUM_WORKERS = 32
N_BLOCKS = N_ROWS // 8
CB = 28
CHUNK_ROWS = CB * 8
NCH = 14


def _body(h_hbm, out_hbm, buf0, buf1, si0, si1, so0, so1):
    c = lax.axis_index("c")
    s = lax.axis_index("s")
    w = s * 2 + c
    bstart = (N_BLOCKS * w) // NUM_WORKERS
    bend = (N_BLOCKS * (w + 1)) // NUM_WORKERS
    bufs = (buf0, buf1)
    sis = (si0, si1)
    sos = (so0, so1)

    def row0_of(k):
        return jnp.minimum(bstart + k * CB, bend - CB) * 8

    def in_copy(b, r0):
        return pltpu.make_async_copy(
            h_hbm.at[pl.ds(r0, CHUNK_ROWS), :], bufs[b], sis[b]
        )

    def out_copy(b, r0):
        return pltpu.make_async_copy(
            bufs[b].at[:, pl.ds(0, OUT_FEATURES)],
            out_hbm.at[pl.ds(r0, CHUNK_ROWS), :],
            sos[b],
        )

    for b in range(2):
        in_copy(b, row0_of(b)).start()

    def step(k, b):
        in_copy(b, row0_of(k)).wait()

        @pl.when(k >= 1)
        def _():
            out_copy(1 - b, row0_of(k - 1)).wait()

        @pl.when(jnp.logical_and(k >= 1, k + 1 < NCH))
        def _():
            in_copy(1 - b, row0_of(k + 1)).start()

        def row_body(r, cc):
            for l in range(1, N_LEVELS):
                bufs[b][r, pl.ds(SEL_W * l, SEL_W)] = bufs[b][
                    r, pl.ds(LEVEL_W * l, SEL_W)
                ]
            return cc

        lax.fori_loop(0, CHUNK_ROWS, row_body, 0, unroll=8)
        out_copy(b, row0_of(k)).start()

    def outer(j, carry):
        step(j * 2, 0)
        step(j * 2 + 1, 1)
        return carry

    lax.fori_loop(0, NCH // 2, outer, 0)
    out_copy(1, row0_of(NCH - 1)).wait()


@jax.jit
def kernel(h):
    mesh = plsc.VectorSubcoreMesh(core_axis_name="c", subcore_axis_name="s")
    return pl.kernel(
        _body,
        out_type=jax.ShapeDtypeStruct((N_ROWS, OUT_FEATURES), jnp.float32),
        mesh=mesh,
        scratch_types=[
            pltpu.VMEM((CHUNK_ROWS, IN_FEATURES), jnp.float32),
            pltpu.VMEM((CHUNK_ROWS, IN_FEATURES), jnp.float32),
            pltpu.SemaphoreType.DMA,
            pltpu.SemaphoreType.DMA,
            pltpu.SemaphoreType.DMA,
            pltpu.SemaphoreType.DMA,
        ],
    )(h)
